# Initial kernel scaffold; baseline (speedup 1.0000x reference)
import jax
import jax.numpy as jnp
from jax.experimental import pallas as pl


def kernel(x, edge_index):
    raise NotImplementedError("write your pallas kernel here")


# speedup vs baseline: 8.9569x; 8.9569x over previous
import functools

import jax
import jax.numpy as jnp
from jax import lax
from jax.experimental import pallas as pl
from jax.experimental.pallas import tpu as pltpu
from jax.experimental.pallas import tpu_sc as plsc

N = 10000
E = 320000
D = 128
DH = D // 2

NC = 2
NS = 16
NP = 10240
ROWS_PER_TILE = NP // NS

CH = 80
EPT = E // NS
NCH = EPT // CH

_mesh = plsc.VectorSubcoreMesh(core_axis_name="c", subcore_axis_name="s")


@functools.partial(
    pl.kernel,
    mesh=_mesh,
    compiler_params=pltpu.CompilerParams(use_tc_tiling_on_sc=False),
    out_type=jax.ShapeDtypeStruct((NC, NP, DH), jnp.float32),
    scratch_types=[
        pltpu.VMEM((NCH, CH), jnp.int32),
        pltpu.VMEM((NCH, CH), jnp.int32),
        pltpu.VMEM((CH, DH), jnp.float32),
        pltpu.VMEM((CH, DH), jnp.float32),
        pltpu.VMEM_SHARED((NP, DH), jnp.float32),
        pltpu.SemaphoreType.DMA,
        pltpu.SemaphoreType.DMA,
    ],
)
def _scatter_sum(x2_hbm, src_hbm, dst_hbm, zeros_hbm, out_hbm,
                 src_v, dst_v, rows0, rows1, acc, sem0, sem1):
    c = lax.axis_index("c")
    s = lax.axis_index("s")

    r0 = s * ROWS_PER_TILE
    pltpu.sync_copy(zeros_hbm.at[pl.ds(r0, ROWS_PER_TILE)],
                    acc.at[pl.ds(r0, ROWS_PER_TILE)])

    pltpu.sync_copy(src_hbm.at[c, s], src_v)
    pltpu.sync_copy(dst_hbm.at[s], dst_v)

    plsc.subcore_barrier()

    def gather(i, buf, sem):
        pltpu.async_copy(x2_hbm.at[src_v.at[i]], buf, sem)

    def gwait(i, buf, sem):
        pltpu.make_async_copy(x2_hbm.at[src_v.at[i]], buf, sem).wait()

    def scat(i, buf):
        pltpu.sync_copy(buf, acc.at[dst_v.at[i]], add=True)

    gather(0, rows0, sem0)

    def body(j, carry):
        ca = 2 * j
        gather(ca + 1, rows1, sem1)
        gwait(ca, rows0, sem0)
        scat(ca, rows0)
        gather(ca + 2, rows0, sem0)
        gwait(ca + 1, rows1, sem1)
        scat(ca + 1, rows1)
        return carry

    lax.fori_loop(0, (NCH - 2) // 2, body, 0)
    gather(NCH - 1, rows1, sem1)
    gwait(NCH - 2, rows0, sem0)
    scat(NCH - 2, rows0)
    gwait(NCH - 1, rows1, sem1)
    scat(NCH - 1, rows1)

    plsc.subcore_barrier()

    pltpu.sync_copy(acc.at[pl.ds(r0, ROWS_PER_TILE)],
                    out_hbm.at[c, pl.ds(r0, ROWS_PER_TILE)])


def _combine_body(x_ref, p_ref, o_ref):
    o_ref[...] = x_ref[...] + jnp.concatenate([p_ref[0], p_ref[1]], axis=-1)


_BN = 1000


def _combine(x, partials):
    return pl.pallas_call(
        _combine_body,
        grid=(N // _BN,),
        in_specs=[
            pl.BlockSpec((_BN, D), lambda i: (i, 0)),
            pl.BlockSpec((NC, _BN, DH), lambda i: (0, i, 0)),
        ],
        out_specs=pl.BlockSpec((_BN, D), lambda i: (i, 0)),
        out_shape=jax.ShapeDtypeStruct((N, D), jnp.float32),
    )(x, partials)


def kernel(x, edge_index):
    x2 = x.reshape(2 * N, DH)
    src = edge_index[0]
    dst = edge_index[1]
    src2 = jnp.stack([2 * src, 2 * src + 1]).reshape(NC, NS, NCH, CH)
    dst3 = dst.reshape(NS, NCH, CH)
    zeros = jnp.zeros((NP, DH), jnp.float32)
    partials = _scatter_sum(x2, src2, dst3, zeros)
    return _combine(x, partials)


# speedup vs baseline: 10.4293x; 1.1644x over previous
import functools

import jax
import jax.numpy as jnp
from jax import lax
from jax.experimental import pallas as pl
from jax.experimental.pallas import tpu as pltpu
from jax.experimental.pallas import tpu_sc as plsc

N = 10000
E = 320000
D = 128
DH = D // 2

NC = 2
NS = 16
NP = 10240
ROWS_PER_TILE = NP // NS

CH = 125
EPT = E // NS
NCH = EPT // CH

_mesh = plsc.VectorSubcoreMesh(core_axis_name="c", subcore_axis_name="s")


@functools.partial(
    pl.kernel,
    mesh=_mesh,
    compiler_params=pltpu.CompilerParams(use_tc_tiling_on_sc=False),
    out_type=jax.ShapeDtypeStruct((NC, NP, DH), jnp.float32),
    scratch_types=[
        pltpu.VMEM((NCH, CH), jnp.int32),
        pltpu.VMEM((NCH, CH), jnp.int32),
        pltpu.VMEM((CH, DH), jnp.float32),
        pltpu.VMEM((CH, DH), jnp.float32),
        pltpu.VMEM_SHARED((NP, DH), jnp.float32),
        pltpu.SemaphoreType.DMA,
        pltpu.SemaphoreType.DMA,
    ],
)
def _scatter_sum(x2_hbm, src_hbm, dst_hbm, zeros_hbm, out_hbm,
                 src_v, dst_v, rows0, rows1, acc, sem0, sem1):
    c = lax.axis_index("c")
    s = lax.axis_index("s")

    r0 = s * ROWS_PER_TILE
    pltpu.sync_copy(zeros_hbm.at[pl.ds(r0, ROWS_PER_TILE)],
                    acc.at[pl.ds(r0, ROWS_PER_TILE)])

    pltpu.sync_copy(src_hbm.at[c, s], src_v)
    pltpu.sync_copy(dst_hbm.at[s], dst_v)

    plsc.subcore_barrier()

    def gather(i, buf, sem):
        pltpu.async_copy(x2_hbm.at[src_v.at[i]], buf, sem)

    def gwait(i, buf, sem):
        pltpu.make_async_copy(x2_hbm.at[src_v.at[i]], buf, sem).wait()

    def scat(i, buf):
        pltpu.sync_copy(buf, acc.at[dst_v.at[i]], add=True)

    gather(0, rows0, sem0)

    def body(j, carry):
        ca = 2 * j
        gather(ca + 1, rows1, sem1)
        gwait(ca, rows0, sem0)
        scat(ca, rows0)
        gather(ca + 2, rows0, sem0)
        gwait(ca + 1, rows1, sem1)
        scat(ca + 1, rows1)
        return carry

    lax.fori_loop(0, (NCH - 2) // 2, body, 0)
    gather(NCH - 1, rows1, sem1)
    gwait(NCH - 2, rows0, sem0)
    scat(NCH - 2, rows0)
    gwait(NCH - 1, rows1, sem1)
    scat(NCH - 1, rows1)

    plsc.subcore_barrier()

    pltpu.sync_copy(acc.at[pl.ds(r0, ROWS_PER_TILE)],
                    out_hbm.at[c, pl.ds(r0, ROWS_PER_TILE)])


def _combine_body(x_ref, p_ref, o_ref):
    o_ref[...] = x_ref[...] + jnp.concatenate([p_ref[0], p_ref[1]], axis=-1)


_BN = 1000


def _combine(x, partials):
    return pl.pallas_call(
        _combine_body,
        grid=(N // _BN,),
        in_specs=[
            pl.BlockSpec((_BN, D), lambda i: (i, 0)),
            pl.BlockSpec((NC, _BN, DH), lambda i: (0, i, 0)),
        ],
        out_specs=pl.BlockSpec((_BN, D), lambda i: (i, 0)),
        out_shape=jax.ShapeDtypeStruct((N, D), jnp.float32),
    )(x, partials)


def kernel(x, edge_index):
    x2 = x.reshape(2 * N, DH)
    src = edge_index[0]
    dst = edge_index[1]
    src2 = jnp.stack([2 * src, 2 * src + 1]).reshape(NC, NS, NCH, CH)
    dst3 = dst.reshape(NS, NCH, CH)
    zeros = jnp.zeros((NP, DH), jnp.float32)
    partials = _scatter_sum(x2, src2, dst3, zeros)
    return _combine(x, partials)


# speedup vs baseline: 11.9011x; 1.1411x over previous
import functools

import jax
import jax.numpy as jnp
from jax import lax
from jax.experimental import pallas as pl
from jax.experimental.pallas import tpu as pltpu
from jax.experimental.pallas import tpu_sc as plsc

N = 10000
E = 320000
D = 128
DH = D // 2

NC = 2
NS = 16
NP = 10240
ROWS_PER_TILE = NP // NS

CH = 125
EPT = E // NS
NCH = EPT // CH

_mesh = plsc.VectorSubcoreMesh(core_axis_name="c", subcore_axis_name="s")


@functools.partial(
    pl.kernel,
    mesh=_mesh,
    compiler_params=pltpu.CompilerParams(use_tc_tiling_on_sc=False),
    out_type=jax.ShapeDtypeStruct((NC, NP, DH), jnp.float32),
    scratch_types=[
        pltpu.VMEM((NCH, CH), jnp.int32),
        pltpu.VMEM((NCH, CH), jnp.int32),
        pltpu.VMEM((CH, DH), jnp.float32),
        pltpu.VMEM((CH, DH), jnp.float32),
        pltpu.VMEM((CH, DH), jnp.float32),
        pltpu.VMEM((CH, DH), jnp.float32),
        pltpu.VMEM_SHARED((NP, DH), jnp.float32),
        pltpu.SemaphoreType.DMA,
        pltpu.SemaphoreType.DMA,
        pltpu.SemaphoreType.DMA,
        pltpu.SemaphoreType.DMA,
        pltpu.SemaphoreType.DMA,
        pltpu.SemaphoreType.DMA,
        pltpu.SemaphoreType.DMA,
        pltpu.SemaphoreType.DMA,
    ],
)
def _scatter_sum(x2_hbm, src_hbm, dst_hbm, zeros_hbm, out_hbm,
                 src_v, dst_v, rows0, rows1, rows2, rows3, acc,
                 gs0, gs1, gs2, gs3, ss0, ss1, ss2, ss3):
    c = lax.axis_index("c")
    s = lax.axis_index("s")

    r0 = s * ROWS_PER_TILE
    pltpu.sync_copy(zeros_hbm.at[pl.ds(r0, ROWS_PER_TILE)],
                    acc.at[pl.ds(r0, ROWS_PER_TILE)])

    pltpu.sync_copy(src_hbm.at[c, s], src_v)
    pltpu.sync_copy(dst_hbm.at[s], dst_v)

    plsc.subcore_barrier()

    bufs = (rows0, rows1, rows2, rows3)
    gsem = (gs0, gs1, gs2, gs3)
    ssem = (ss0, ss1, ss2, ss3)

    def gissue(i, b):
        pltpu.async_copy(x2_hbm.at[src_v.at[i]], bufs[b], gsem[b])

    def gwait(i, b):
        pltpu.make_async_copy(x2_hbm.at[src_v.at[i]], bufs[b], gsem[b]).wait()

    def sissue(i, b):
        pltpu.async_copy(bufs[b], acc.at[dst_v.at[i]], ssem[b], add=True)

    def swait(i, b):
        pltpu.make_async_copy(bufs[b], acc.at[dst_v.at[i]], ssem[b]).wait()

    gissue(0, 0)
    gissue(1, 1)
    gissue(2, 2)
    gwait(0, 0)
    sissue(0, 0)
    gissue(3, 3)
    gwait(1, 1)
    sissue(1, 1)

    def body(j, carry):
        c4 = 4 * j + 2
        for b in range(4):
            ci = c4 + b
            swait(ci - 2, b)
            gissue(ci + 2, b)
            gwait(ci, (2 + b) % 4)
            sissue(ci, (2 + b) % 4)
        return carry

    lax.fori_loop(0, (NCH - 4) // 4, body, 0)
    swait(NCH - 4, (NCH - 4) % 4)
    gwait(NCH - 2, (NCH - 2) % 4)
    sissue(NCH - 2, (NCH - 2) % 4)
    swait(NCH - 3, (NCH - 3) % 4)
    gwait(NCH - 1, (NCH - 1) % 4)
    sissue(NCH - 1, (NCH - 1) % 4)
    swait(NCH - 2, (NCH - 2) % 4)
    swait(NCH - 1, (NCH - 1) % 4)

    plsc.subcore_barrier()

    pltpu.sync_copy(acc.at[pl.ds(r0, ROWS_PER_TILE)],
                    out_hbm.at[c, pl.ds(r0, ROWS_PER_TILE)])


def _combine_body(x_ref, p_ref, o_ref):
    o_ref[...] = x_ref[...] + jnp.concatenate([p_ref[0], p_ref[1]], axis=-1)


_BN = 1000


def _combine(x, partials):
    return pl.pallas_call(
        _combine_body,
        grid=(N // _BN,),
        in_specs=[
            pl.BlockSpec((_BN, D), lambda i: (i, 0)),
            pl.BlockSpec((NC, _BN, DH), lambda i: (0, i, 0)),
        ],
        out_specs=pl.BlockSpec((_BN, D), lambda i: (i, 0)),
        out_shape=jax.ShapeDtypeStruct((N, D), jnp.float32),
    )(x, partials)


def kernel(x, edge_index):
    x2 = x.reshape(2 * N, DH)
    src = edge_index[0]
    dst = edge_index[1]
    src2 = jnp.stack([2 * src, 2 * src + 1]).reshape(NC, NS, NCH, CH)
    dst3 = dst.reshape(NS, NCH, CH)
    zeros = jnp.zeros((NP, DH), jnp.float32)
    partials = _scatter_sum(x2, src2, dst3, zeros)
    return _combine(x, partials)
